# Initial kernel scaffold; baseline (speedup 1.0000x reference)
import jax
import jax.numpy as jnp
from jax.experimental import pallas as pl


def kernel(x, edge_index, W1, b1, g1, be1, W2, b2):
    raise NotImplementedError("write your pallas kernel here")


# speedup vs baseline: 4.3176x; 4.3176x over previous
import functools

import jax
import jax.numpy as jnp
from jax import lax
from jax.experimental import pallas as pl
from jax.experimental.pallas import tpu as pltpu
from jax.experimental.pallas import tpu_sc as plsc

N_NODES = 10000
N_EDGES = 320000
D = 128

NC = 2
NS = 16
NW = NC * NS

CHUNK = 128
N_PAD = 10240
ROWS_PER_TILE = N_PAD // NS
E_DIR = 2 * N_EDGES
CHUNKS_PER_W = -(-E_DIR // (NW * CHUNK))
E_PAD = NW * CHUNKS_PER_W * CHUNK


def _sc_messages(x_pad, src, dst, zeros_tile):
    mesh = plsc.VectorSubcoreMesh(core_axis_name="c", subcore_axis_name="s")

    @functools.partial(
        pl.kernel,
        out_type=jax.ShapeDtypeStruct((NC, N_PAD, D), jnp.float32),
        mesh=mesh,
        scratch_types=[
            pltpu.VMEM((CHUNK,), jnp.int32),
            pltpu.VMEM((CHUNK,), jnp.int32),
            pltpu.VMEM((CHUNK, D), jnp.float32),
            pltpu.VMEM_SHARED((N_PAD, D), jnp.float32),
        ],
    )
    def body(x_hbm, src_hbm, dst_hbm, zero_hbm, out_hbm, idx_s, idx_d, rows, acc):
        cid = lax.axis_index("c")
        sid = lax.axis_index("s")
        wid = sid * NC + cid

        pltpu.sync_copy(zero_hbm, acc.at[pl.ds(sid * ROWS_PER_TILE, ROWS_PER_TILE)])
        plsc.subcore_barrier()

        @pl.loop(0, CHUNKS_PER_W)
        def _(c):
            pltpu.sync_copy(src_hbm.at[wid, c], idx_s)
            pltpu.sync_copy(dst_hbm.at[wid, c], idx_d)
            pltpu.sync_copy(x_hbm.at[idx_s], rows)
            pltpu.sync_copy(rows, acc.at[idx_d], add=True)

        plsc.subcore_barrier()
        pltpu.sync_copy(
            acc.at[pl.ds(sid * ROWS_PER_TILE, ROWS_PER_TILE)],
            out_hbm.at[cid, pl.ds(sid * ROWS_PER_TILE, ROWS_PER_TILE)],
        )

    return body(x_pad, src, dst, zeros_tile)


def _tc_mlp(msgs, x_pad, W1a, W1b, b1, g1, be1, W2, b2):
    BN = 2048

    def body(m_ref, x_ref, w1a_ref, w1b_ref, b1_ref, g1_ref, be1_ref,
             w2_ref, b2_ref, o_ref):
        m = m_ref[0] + m_ref[1]
        h = (
            jnp.dot(m, w1a_ref[...], preferred_element_type=jnp.float32,
                    precision=lax.Precision.HIGHEST)
            + jnp.dot(x_ref[...], w1b_ref[...], preferred_element_type=jnp.float32,
                      precision=lax.Precision.HIGHEST)
            + b1_ref[...]
        )
        mu = jnp.mean(h, axis=-1, keepdims=True)
        var = jnp.mean((h - mu) ** 2, axis=-1, keepdims=True)
        h = (h - mu) * lax.rsqrt(var + 1e-5) * g1_ref[...] + be1_ref[...]
        h = jnp.tanh(h)
        o_ref[...] = (
            jnp.dot(h, w2_ref[...], preferred_element_type=jnp.float32,
                    precision=lax.Precision.HIGHEST)
            + b2_ref[...]
        )

    full = lambda shape: pl.BlockSpec(shape, lambda i: tuple(0 for _ in shape))
    return pl.pallas_call(
        body,
        grid=(N_PAD // BN,),
        in_specs=[
            pl.BlockSpec((NC, BN, D), lambda i: (0, i, 0)),
            pl.BlockSpec((BN, D), lambda i: (i, 0)),
            full((D, D)),
            full((D, D)),
            full((1, D)),
            full((1, D)),
            full((1, D)),
            full((D, D)),
            full((1, D)),
        ],
        out_specs=pl.BlockSpec((BN, D), lambda i: (i, 0)),
        out_shape=jax.ShapeDtypeStruct((N_PAD, D), jnp.float32),
    )(msgs, x_pad, W1a, W1b, b1, g1, be1, W2, b2)


def kernel(x, edge_index, W1, b1, g1, be1, W2, b2):
    x_pad = jnp.pad(x, ((0, N_PAD - N_NODES), (0, 0)))
    s = edge_index[0]
    e = edge_index[1]
    n_fill = E_PAD - E_DIR
    src = jnp.concatenate([s, e, jnp.zeros((n_fill,), jnp.int32)])
    dst = jnp.concatenate([e, s, jnp.full((n_fill,), N_NODES, jnp.int32)])
    src = src.reshape(NW, CHUNKS_PER_W, CHUNK)
    dst = dst.reshape(NW, CHUNKS_PER_W, CHUNK)
    zeros_tile = jnp.zeros((ROWS_PER_TILE, D), jnp.float32)

    msgs = _sc_messages(x_pad, src, dst, zeros_tile)
    out = _tc_mlp(msgs, x_pad, W1[:D], W1[D:], b1.reshape(1, D),
                  g1.reshape(1, D), be1.reshape(1, D), W2, b2.reshape(1, D))
    return out[:N_NODES]
